# baseline (speedup 1.0000x reference)
import functools

import jax
import jax.numpy as jnp
from jax.experimental import pallas as pl
from jax.experimental.pallas import tpu as pltpu
from jax.experimental.pallas import tpu_sc as plsc

VOCAB = 1000000
DIM = 64
BATCH = 16384
HIST = 20
N = BATCH * HIST

WINDOW = 128
ROWS_PER_BLK = 4
GRID = N // (WINDOW * ROWS_PER_BLK)

_mesh = plsc.VectorSubcoreMesh(core_axis_name="core", subcore_axis_name="subcore")


@jax.jit
def _gather(table2d, idx2d):
    @functools.partial(
        pl.kernel,
        out_type=jax.ShapeDtypeStruct((N, DIM), table2d.dtype),
        mesh=_mesh,
        scratch_types=[pltpu.SemaphoreType.DMA],
        compiler_params=pltpu.CompilerParams(use_tc_tiling_on_sc=False),
    )
    def kern(table_hbm, idx_hbm, out_hbm, sem):
        def body(i_vmem, o_vmem):
            copies = [
                pltpu.async_copy(
                    table_hbm.at[i_vmem.at[j, :]],
                    o_vmem.at[pl.ds(j * WINDOW, WINDOW)],
                    sem,
                )
                for j in range(ROWS_PER_BLK)
            ]
            for c in copies:
                c.wait()

        pltpu.emit_pipeline(
            body,
            grid=(GRID,),
            in_specs=[pl.BlockSpec((ROWS_PER_BLK, WINDOW), index_map=lambda g: (g, 0))],
            out_specs=[
                pl.BlockSpec(
                    (ROWS_PER_BLK * WINDOW, DIM), index_map=lambda g: (g, 0)
                )
            ],
            core_axis_name=("core", "subcore"),
            dimension_semantics=(pltpu.PARALLEL,),
        )(idx_hbm, out_hbm)

    return kern(table2d, idx2d)


def kernel(inputs, table):
    out2d = _gather(table, jnp.reshape(inputs, (N // WINDOW, WINDOW)))
    return out2d.reshape(BATCH, HIST, DIM)
